# baseline (speedup 1.0000x reference)
import functools

import jax
import jax.numpy as jnp
import numpy as np
from jax import lax
import jax.experimental.pallas as pl
from jax.experimental.pallas import tpu as pltpu

_DIM = 96
_NE = 5
_IMG = 64
_HW = _IMG * _IMG
_PAD = 2 * _IMG
_SCR = _HW + 2 * _PAD
_INV_SQRT2 = np.float32(0.7071067811865476)
_BN_S = np.float32(1.0 / np.sqrt(1.0 + 1e-5))
_D4 = _DIM // 4


def _gelu(v):
    return 0.5 * v * (1.0 + lax.erf(v * _INV_SQRT2))


def _mmt(a, w_ref, b_ref):
    r = lax.dot_general(a.astype(jnp.bfloat16),
                        w_ref[...].astype(jnp.bfloat16),
                        (((1,), (1,)), ((), ())),
                        preferred_element_type=jnp.float32)
    return r + b_ref[...] if b_ref is not None else r


def _router_kernel(nb, x_ref, w1_ref, b1_ref, w2_ref, b2_ref, w_ref,
                   pooled_scr):
    b = pl.program_id(0)
    pooled = jnp.mean(x_ref[0], axis=1, keepdims=True)
    for bi in range(nb):
        @pl.when(b == bi)
        def _():
            pooled_scr[:, bi:bi + 1] = pooled

    @pl.when(b == nb - 1)
    def _():
        ii = lax.broadcasted_iota(jnp.int32, (_DIM, _DIM), 0)
        jj = lax.broadcasted_iota(jnp.int32, (_DIM, _DIM), 1)
        eye = (ii == jj).astype(jnp.float32)
        pooled_t = lax.dot_general(
            pooled_scr[:, 0:nb], eye, (((0,), (0,)), ((), ())),
            preferred_element_type=jnp.float32)
        h = _gelu(_mmt(pooled_t, w1_ref, b1_ref))
        logits = _mmt(h, w2_ref, b2_ref)
        cols = []
        for e in range(_NE):
            ce = logits[:, e:e + 1]
            rank = jnp.sum(jnp.where(logits > ce, 1.0, 0.0),
                           axis=1, keepdims=True)
            for j in range(e):
                rank = rank + jnp.where(logits[:, j:j + 1] == ce, 1.0, 0.0)
            cols.append(rank)
        sel = jnp.concatenate(cols, axis=1) < 2.5
        lm = jnp.where(sel, logits, jnp.float32(-1e30))
        m = jnp.max(lm, axis=1, keepdims=True)
        ex = jnp.where(sel, jnp.exp(logits - m), 0.0)
        w_ref[...] = ex / jnp.sum(ex, axis=1, keepdims=True)


def _moe_kernel(treedef, w_ref, xf_ref, *args):
    out_ref, acc_ref = args[-7], args[-6]
    scr = args[-5:]
    P = jax.tree_util.tree_unflatten(treedef, args[:-7])
    b = pl.program_id(0)

    ii = lax.broadcasted_iota(jnp.int32, (_DIM, _DIM), 0)
    jj = lax.broadcasted_iota(jnp.int32, (_DIM, _DIM), 1)
    eye = (ii == jj).astype(jnp.float32)
    x = lax.dot_general(xf_ref[0], eye, (((0,), (0,)), ((), ())),
                        preferred_element_type=jnp.float32)

    row = lax.broadcasted_iota(jnp.int32, (_HW, 1), 0)
    wcol = lax.bitwise_and(row, _IMG - 1)
    zpad = jnp.zeros((_PAD, _DIM), jnp.float32)
    for dw in (-2, -1, 0, 1, 2):
        sref = scr[dw + 2]
        sref[0:_PAD, :] = zpad
        sref[_PAD + _HW:_SCR, :] = zpad
        if dw == 0:
            sref[_PAD:_PAD + _HW, :] = x
        else:
            r = jnp.roll(x, -dw, axis=0)
            m = (wcol < _IMG - dw) if dw > 0 else (wcol >= -dw)
            sref[_PAD:_PAD + _HW, :] = jnp.where(m, r, 0.0)

    def tap(dh, dw):
        base = _PAD + _IMG * dh
        return scr[dw + 2][base:base + _HW, :]

    def ln_lanes(v, g_ref, be_ref):
        mu = jnp.mean(v, axis=1, keepdims=True)
        var = jnp.mean((v - mu) * (v - mu), axis=1, keepdims=True)
        return (v - mu) * lax.rsqrt(var + 1e-6) * g_ref[...] + be_ref[...]

    def att_fuse(feats, q):
        pooled = jnp.mean(feats, axis=0, keepdims=True)
        a = _gelu(_mmt(pooled, q['att_W1'], q['att_b1']))
        a = jax.nn.sigmoid(_mmt(a, q['att_W2'], q['att_b2']))
        g = _mmt(feats * a, q['fus_W'], q['fus_b'])
        return _gelu(ln_lanes(g, q['fus_g'], q['fus_be']))

    def branches4(ts, q):
        wall = jnp.concatenate(
            [q['b' + str(k)]['W'][...] for k in range(4)], axis=0)
        acc = None
        wall = wall.astype(jnp.bfloat16)
        for k in range(4):
            mask = (ii >= k * _D4) & (ii < (k + 1) * _D4)
            wk = jnp.where(mask, wall, jnp.bfloat16(0.0))
            t = lax.dot_general(ts[k].astype(jnp.bfloat16), wk,
                                (((1,), (1,)), ((), ())),
                                preferred_element_type=jnp.float32)
            acc = t if acc is None else acc + t
        s = jnp.concatenate(
            [q['b' + str(k)]['g'][...] for k in range(4)], axis=1) * _BN_S
        bb = jnp.concatenate(
            [q['b' + str(k)]['b'][...] for k in range(4)], axis=1) * s \
            + jnp.concatenate(
            [q['b' + str(k)]['be'][...] for k in range(4)], axis=1)
        return _gelu(acc * s + bb)

    def attn_expert():
        q = P['attn']
        return x + _gelu(_mmt(x, q['W'], q['b']))

    def edge_expert():
        q = P['edge']
        sh = ((tap(-1, 1) - tap(-1, -1)) + 2.0 * (tap(0, 1) - tap(0, -1))
              + (tap(1, 1) - tap(1, -1)))
        sv = ((tap(1, -1) + 2.0 * tap(1, 0) + tap(1, 1))
              - (tap(-1, -1) + 2.0 * tap(-1, 0) + tap(-1, 1)))
        lapv = tap(-1, 0) + tap(0, -1) + tap(0, 1) + tap(1, 0) - 4.0 * x
        d1 = tap(-1, -1) - tap(-1, 1) - tap(1, -1) + tap(1, 1)
        sobel = jnp.sqrt(sh * sh + sv * sv + 1e-08)
        lapE = jnp.abs(lapv)
        diag = jnp.abs(d1)
        gmag = jnp.sqrt(sobel * sobel + lapE * lapE + 1e-08)
        feats = branches4((sobel, lapE, diag, gmag), q)
        return att_fuse(feats, q) + x

    def freq_expert():
        q = P['freq']
        s8 = None
        for dh in (-1, 0, 1):
            for dw in (-1, 0, 1):
                if (dh, dw) == (0, 0):
                    continue
                t = tap(dh, dw)
                s8 = t if s8 is None else s8 + t
        souter = None
        for dh in (-2, -1, 0, 1, 2):
            for dw in (-2, -1, 0, 1, 2):
                if max(abs(dh), abs(dw)) != 2:
                    continue
                t = tap(dh, dw)
                souter = t if souter is None else souter + t
        low = (x + s8) * np.float32(1.0 / 9.0)
        avg5 = (x + s8 + souter) * np.float32(1.0 / 25.0)
        mid = low - avg5
        high = x - low
        feats = branches4((low, mid, high, x), q)
        return att_fuse(feats, q) + x

    def hybrid_expert():
        q = P['hybrid']
        dwt = lax.dot_general(
            q['dw'][...], eye, (((0,), (0,)), ((), ())),
            preferred_element_type=jnp.float32)
        acc = None
        for i in range(5):
            for j in range(5):
                t = dwt[i * 5 + j:i * 5 + j + 1, :] * tap(i - 2, j - 2)
                acc = t if acc is None else acc + t
        h = ln_lanes(acc, q['ln_g'], q['ln_b'])
        return x + _gelu(_mmt(h, q['pw_W'], q['pw_b']))

    def texture_expert():
        q = P['texture']
        dwt = lax.dot_general(
            q['dw'][...], eye, (((0,), (0,)), ((), ())),
            preferred_element_type=jnp.float32)
        acc = None
        for i in range(3):
            for j in range(3):
                t = dwt[i * 3 + j:i * 3 + j + 1, :] * tap(i - 1, j - 1)
                acc = t if acc is None else acc + t
        return x + _mmt(_gelu(acc), q['pw_W'], q['pw_b'])

    acc_ref[...] = jnp.zeros((_HW, _DIM), jnp.float32)

    def gate(e, fn):
        w = w_ref[b, e]

        @pl.when(w > 0.0)
        def _():
            acc_ref[...] += w * fn()

    gate(0, attn_expert)
    gate(1, edge_expert)
    gate(2, hybrid_expert)
    gate(3, freq_expert)
    gate(4, texture_expert)

    out_ref[0] = lax.dot_general(eye, acc_ref[...], (((1,), (1,)), ((), ())),
                                 preferred_element_type=jnp.float32)


def _row(v):
    return v.reshape(1, -1)


@jax.jit
def kernel(x, params):
    B, C, Hh, Ww = x.shape
    xf = x.reshape(B, C, Hh * Ww)

    def fullspec(a):
        return pl.BlockSpec(a.shape, lambda bi, _n=a.ndim: (0,) * _n)

    r = params['router']
    rops = [r['g_W1'], _row(r['g_b1']), r['g_W2'], _row(r['g_b2'])]
    weights = pl.pallas_call(
        functools.partial(_router_kernel, B),
        grid=(B,),
        in_specs=[pl.BlockSpec((1, C, _HW), lambda bi: (bi, 0, 0))]
                 + [fullspec(a) for a in rops],
        out_specs=pl.BlockSpec((B, _NE), lambda bi: (0, 0)),
        out_shape=jax.ShapeDtypeStruct((B, _NE), jnp.float32),
        scratch_shapes=[pltpu.VMEM((_DIM, 8), jnp.float32)],
    )(xf, *rops)

    def cf(p):
        return {
            'b0': {k2: (_row(v) if v.ndim == 1 else v)
                   for k2, v in p['b0'].items()},
            'b1': {k2: (_row(v) if v.ndim == 1 else v)
                   for k2, v in p['b1'].items()},
            'b2': {k2: (_row(v) if v.ndim == 1 else v)
                   for k2, v in p['b2'].items()},
            'b3': {k2: (_row(v) if v.ndim == 1 else v)
                   for k2, v in p['b3'].items()},
            'att_W1': p['att_W1'], 'att_b1': _row(p['att_b1']),
            'att_W2': p['att_W2'], 'att_b2': _row(p['att_b2']),
            'fus_W': p['fus_W'], 'fus_b': _row(p['fus_b']),
            'fus_g': _row(p['fus_g']), 'fus_be': _row(p['fus_be']),
        }

    packed = {
        'attn': {'W': params['attn']['W'], 'b': _row(params['attn']['b'])},
        'edge': cf(params['edge']),
        'freq': cf(params['freq']),
        'hybrid': {
            'dw': params['hybrid']['dw'].reshape(C, 25),
            'ln_g': _row(params['hybrid']['ln_g']),
            'ln_b': _row(params['hybrid']['ln_b']),
            'pw_W': params['hybrid']['pw_W'],
            'pw_b': _row(params['hybrid']['pw_b']),
        },
        'texture': {
            'dw': params['texture']['dw'].reshape(C, 9),
            'pw_W': params['texture']['pw_W'],
            'pw_b': _row(params['texture']['pw_b']),
        },
    }
    leaves, treedef = jax.tree_util.tree_flatten(packed)

    out = pl.pallas_call(
        functools.partial(_moe_kernel, treedef),
        grid=(B,),
        in_specs=[pl.BlockSpec(memory_space=pltpu.SMEM),
                  pl.BlockSpec((1, C, _HW), lambda bi: (bi, 0, 0))]
                 + [fullspec(a) for a in leaves],
        out_specs=pl.BlockSpec((1, C, _HW), lambda bi: (bi, 0, 0)),
        out_shape=jax.ShapeDtypeStruct((B, C, _HW), jnp.float32),
        scratch_shapes=[pltpu.VMEM((_HW, _DIM), jnp.float32)]
                       + [pltpu.VMEM((_SCR, _DIM), jnp.float32)
                          for _ in range(5)],
    )(weights, xf, *leaves)
    return out.reshape(B, C, Hh, Ww)


# speedup vs baseline: 1.0171x; 1.0171x over previous
import functools

import jax
import jax.numpy as jnp
import numpy as np
from jax import lax
import jax.experimental.pallas as pl
from jax.experimental.pallas import tpu as pltpu

_DIM = 96
_NE = 5
_IMG = 64
_HW = _IMG * _IMG
_PAD = 2 * _IMG
_SCR = _HW + 2 * _PAD
_INV_SQRT2 = np.float32(0.7071067811865476)
_BN_S = np.float32(1.0 / np.sqrt(1.0 + 1e-5))
_D4 = _DIM // 4


def _gelu(v):
    return 0.5 * v * (1.0 + lax.erf(v * _INV_SQRT2))


def _mmt(a, w_ref, b_ref):
    r = lax.dot_general(a, w_ref[...], (((1,), (1,)), ((), ())),
                        preferred_element_type=jnp.float32)
    return r + b_ref[...] if b_ref is not None else r


def _router_kernel(nb, x_ref, w1_ref, b1_ref, w2_ref, b2_ref, w_ref,
                   pooled_scr):
    b = pl.program_id(0)
    pooled = jnp.mean(x_ref[0], axis=1, keepdims=True)
    for bi in range(nb):
        @pl.when(b == bi)
        def _():
            pooled_scr[:, bi:bi + 1] = pooled

    @pl.when(b == nb - 1)
    def _():
        ii = lax.broadcasted_iota(jnp.int32, (_DIM, _DIM), 0)
        jj = lax.broadcasted_iota(jnp.int32, (_DIM, _DIM), 1)
        eye = (ii == jj).astype(jnp.float32)
        pooled_t = lax.dot_general(
            pooled_scr[:, 0:nb], eye, (((0,), (0,)), ((), ())),
            preferred_element_type=jnp.float32)
        h = _gelu(_mmt(pooled_t, w1_ref, b1_ref))
        logits = _mmt(h, w2_ref, b2_ref)
        cols = []
        for e in range(_NE):
            ce = logits[:, e:e + 1]
            rank = jnp.sum(jnp.where(logits > ce, 1.0, 0.0),
                           axis=1, keepdims=True)
            for j in range(e):
                rank = rank + jnp.where(logits[:, j:j + 1] == ce, 1.0, 0.0)
            cols.append(rank)
        sel = jnp.concatenate(cols, axis=1) < 2.5
        lm = jnp.where(sel, logits, jnp.float32(-1e30))
        m = jnp.max(lm, axis=1, keepdims=True)
        ex = jnp.where(sel, jnp.exp(logits - m), 0.0)
        w_ref[...] = ex / jnp.sum(ex, axis=1, keepdims=True)


def _moe_kernel(treedef, w_ref, xf_ref, *args):
    out_ref, acc_ref = args[-9], args[-8]
    scr = args[-7:-2]
    tmp = args[-2:]
    P = jax.tree_util.tree_unflatten(treedef, args[:-9])
    b = pl.program_id(0)

    ii = lax.broadcasted_iota(jnp.int32, (_DIM, _DIM), 0)
    jj = lax.broadcasted_iota(jnp.int32, (_DIM, _DIM), 1)
    eye = (ii == jj).astype(jnp.float32)
    x = lax.dot_general(xf_ref[0], eye, (((0,), (0,)), ((), ())),
                        preferred_element_type=jnp.float32)

    row = lax.broadcasted_iota(jnp.int32, (_HW, 1), 0)
    wcol = lax.bitwise_and(row, _IMG - 1)

    @pl.when(b == 0)
    def _():
        zpad = jnp.zeros((_PAD, _DIM), jnp.float32)
        for sref in list(scr) + list(tmp):
            sref[0:_PAD, :] = zpad
            sref[_PAD + _HW:_SCR, :] = zpad

    for dw in (-2, -1, 0, 1, 2):
        sref = scr[dw + 2]
        if dw == 0:
            sref[_PAD:_PAD + _HW, :] = x
        else:
            r = jnp.roll(x, -dw, axis=0)
            m = (wcol < _IMG - dw) if dw > 0 else (wcol >= -dw)
            sref[_PAD:_PAD + _HW, :] = jnp.where(m, r, 0.0)

    def tap(dh, dw):
        base = _PAD + _IMG * dh
        return scr[dw + 2][base:base + _HW, :]

    def bank(v, sref):
        sref[_PAD:_PAD + _HW, :] = v

        def rd(dh):
            base = _PAD + _IMG * dh
            return sref[base:base + _HW, :]
        return rd

    def ln_lanes(v, g_ref, be_ref):
        mu = jnp.mean(v, axis=1, keepdims=True)
        var = jnp.mean((v - mu) * (v - mu), axis=1, keepdims=True)
        return (v - mu) * lax.rsqrt(var + 1e-6) * g_ref[...] + be_ref[...]

    def att_fuse(feats, q):
        pooled = jnp.mean(feats, axis=0, keepdims=True)
        a = _gelu(_mmt(pooled, q['att_W1'], q['att_b1']))
        a = jax.nn.sigmoid(_mmt(a, q['att_W2'], q['att_b2']))
        g = _mmt(feats * a, q['fus_W'], q['fus_b'])
        return _gelu(ln_lanes(g, q['fus_g'], q['fus_be']))

    def branches4(ts, q):
        wall = jnp.concatenate(
            [q['b' + str(k)]['W'][...] for k in range(4)], axis=0)
        acc = None
        for k in range(4):
            mask = (ii >= k * _D4) & (ii < (k + 1) * _D4)
            wk = jnp.where(mask, wall, 0.0)
            t = lax.dot_general(ts[k], wk, (((1,), (1,)), ((), ())),
                                preferred_element_type=jnp.float32)
            acc = t if acc is None else acc + t
        s = jnp.concatenate(
            [q['b' + str(k)]['g'][...] for k in range(4)], axis=1) * _BN_S
        bb = jnp.concatenate(
            [q['b' + str(k)]['b'][...] for k in range(4)], axis=1) * s \
            + jnp.concatenate(
            [q['b' + str(k)]['be'][...] for k in range(4)], axis=1)
        return _gelu(acc * s + bb)

    def attn_expert():
        q = P['attn']
        return _gelu(_mmt(x, q['W'], q['b']))

    def edge_expert():
        q = P['edge']
        cw = tap(0, 1) - tap(0, -1)
        rdh = bank(cw, tmp[0])
        sh = rdh(-1) + 2.0 * cw + rdh(1)
        cs = tap(0, -1) + 2.0 * x + tap(0, 1)
        rds = bank(cs, tmp[1])
        sv = rds(1) - rds(-1)
        lapv = tap(-1, 0) + tap(0, -1) + tap(0, 1) + tap(1, 0) - 4.0 * x
        cd = tap(0, -1) - tap(0, 1)
        rdd = bank(cd, tmp[0])
        d1 = rdd(-1) - rdd(1)
        sobel = jnp.sqrt(sh * sh + sv * sv + 1e-08)
        lapE = jnp.abs(lapv)
        diag = jnp.abs(d1)
        gmag = jnp.sqrt(sobel * sobel + lapE * lapE + 1e-08)
        feats = branches4((sobel, lapE, diag, gmag), q)
        return att_fuse(feats, q)

    def freq_expert():
        q = P['freq']
        cs3 = tap(0, -1) + x + tap(0, 1)
        rd3 = bank(cs3, tmp[0])
        s9 = rd3(-1) + cs3 + rd3(1)
        cs5 = cs3 + tap(0, -2) + tap(0, 2)
        rd5 = bank(cs5, tmp[1])
        s25 = rd5(-2) + rd5(-1) + cs5 + rd5(1) + rd5(2)
        low = s9 * np.float32(1.0 / 9.0)
        avg5 = s25 * np.float32(1.0 / 25.0)
        mid = low - avg5
        high = x - low
        feats = branches4((low, mid, high, x), q)
        return att_fuse(feats, q)

    def hybrid_expert():
        q = P['hybrid']
        dwt = lax.dot_general(
            q['dw'][...], eye, (((0,), (0,)), ((), ())),
            preferred_element_type=jnp.float32)
        acc = None
        for i in range(5):
            for j in range(5):
                t = dwt[i * 5 + j:i * 5 + j + 1, :] * tap(i - 2, j - 2)
                acc = t if acc is None else acc + t
        h = ln_lanes(acc, q['ln_g'], q['ln_b'])
        return _gelu(_mmt(h, q['pw_W'], q['pw_b']))

    def texture_expert():
        q = P['texture']
        dwt = lax.dot_general(
            q['dw'][...], eye, (((0,), (0,)), ((), ())),
            preferred_element_type=jnp.float32)
        acc = None
        for i in range(3):
            for j in range(3):
                t = dwt[i * 3 + j:i * 3 + j + 1, :] * tap(i - 1, j - 1)
                acc = t if acc is None else acc + t
        return _mmt(_gelu(acc), q['pw_W'], q['pw_b'])

    acc_ref[...] = x

    def gate(e, fn):
        w = w_ref[b, e]

        @pl.when(w > 0.0)
        def _():
            acc_ref[...] += w * fn()

    gate(0, attn_expert)
    gate(1, edge_expert)
    gate(2, hybrid_expert)
    gate(3, freq_expert)
    gate(4, texture_expert)

    out_ref[0] = lax.dot_general(eye, acc_ref[...], (((1,), (1,)), ((), ())),
                                 preferred_element_type=jnp.float32)


def _row(v):
    return v.reshape(1, -1)


@jax.jit
def kernel(x, params):
    B, C, Hh, Ww = x.shape
    xf = x.reshape(B, C, Hh * Ww)

    def fullspec(a):
        return pl.BlockSpec(a.shape, lambda bi, _n=a.ndim: (0,) * _n)

    r = params['router']
    rops = [r['g_W1'], _row(r['g_b1']), r['g_W2'], _row(r['g_b2'])]
    weights = pl.pallas_call(
        functools.partial(_router_kernel, B),
        grid=(B,),
        in_specs=[pl.BlockSpec((1, C, _HW), lambda bi: (bi, 0, 0))]
                 + [fullspec(a) for a in rops],
        out_specs=pl.BlockSpec((B, _NE), lambda bi: (0, 0)),
        out_shape=jax.ShapeDtypeStruct((B, _NE), jnp.float32),
        scratch_shapes=[pltpu.VMEM((_DIM, 8), jnp.float32)],
    )(xf, *rops)

    def cf(p):
        return {
            'b0': {k2: (_row(v) if v.ndim == 1 else v)
                   for k2, v in p['b0'].items()},
            'b1': {k2: (_row(v) if v.ndim == 1 else v)
                   for k2, v in p['b1'].items()},
            'b2': {k2: (_row(v) if v.ndim == 1 else v)
                   for k2, v in p['b2'].items()},
            'b3': {k2: (_row(v) if v.ndim == 1 else v)
                   for k2, v in p['b3'].items()},
            'att_W1': p['att_W1'], 'att_b1': _row(p['att_b1']),
            'att_W2': p['att_W2'], 'att_b2': _row(p['att_b2']),
            'fus_W': p['fus_W'], 'fus_b': _row(p['fus_b']),
            'fus_g': _row(p['fus_g']), 'fus_be': _row(p['fus_be']),
        }

    packed = {
        'attn': {'W': params['attn']['W'], 'b': _row(params['attn']['b'])},
        'edge': cf(params['edge']),
        'freq': cf(params['freq']),
        'hybrid': {
            'dw': params['hybrid']['dw'].reshape(C, 25),
            'ln_g': _row(params['hybrid']['ln_g']),
            'ln_b': _row(params['hybrid']['ln_b']),
            'pw_W': params['hybrid']['pw_W'],
            'pw_b': _row(params['hybrid']['pw_b']),
        },
        'texture': {
            'dw': params['texture']['dw'].reshape(C, 9),
            'pw_W': params['texture']['pw_W'],
            'pw_b': _row(params['texture']['pw_b']),
        },
    }
    leaves, treedef = jax.tree_util.tree_flatten(packed)

    out = pl.pallas_call(
        functools.partial(_moe_kernel, treedef),
        grid=(B,),
        in_specs=[pl.BlockSpec(memory_space=pltpu.SMEM),
                  pl.BlockSpec((1, C, _HW), lambda bi: (bi, 0, 0))]
                 + [fullspec(a) for a in leaves],
        out_specs=pl.BlockSpec((1, C, _HW), lambda bi: (bi, 0, 0)),
        out_shape=jax.ShapeDtypeStruct((B, C, _HW), jnp.float32),
        scratch_shapes=[pltpu.VMEM((_HW, _DIM), jnp.float32)]
                       + [pltpu.VMEM((_SCR, _DIM), jnp.float32)
                          for _ in range(7)],
    )(weights, xf, *leaves)
    return out.reshape(B, C, Hh, Ww)
